# baseline (speedup 1.0000x reference)
import functools

import jax
import jax.numpy as jnp
from jax import lax
from jax.experimental import pallas as pl
from jax.experimental.pallas import tpu as pltpu
from jax.experimental.pallas import tpu_sc as plsc

BATCH = 4
NUM_VIRTUAL_TOKENS = 64
NUM_TOKENS = BATCH * NUM_VIRTUAL_TOKENS
ROW_DIM = 98304
F = 64
DC = ROW_DIM // F
NC, NS = 2, 16
NW = NC * NS
TPW = NUM_TOKENS // NW
GPT = F // 16


def _make_kernel():
    mesh = plsc.VectorSubcoreMesh(core_axis_name="c", subcore_axis_name="s")

    @functools.partial(
        pl.kernel,
        mesh=mesh,
        out_type=jax.ShapeDtypeStruct((NUM_TOKENS * F, DC), jnp.float32),
        compiler_params=pltpu.CompilerParams(needs_layout_passes=False),
        scratch_types=[
            pltpu.VMEM((NUM_TOKENS,), jnp.int32),
            pltpu.VMEM((16, DC), jnp.float32),
            pltpu.SemaphoreType.DMA,
        ],
    )
    def gather_kernel(prefix_hbm, table_hbm, out_hbm, pvals, buf, gsem):
        wid = lax.axis_index("s") * NC + lax.axis_index("c")
        pltpu.sync_copy(prefix_hbm, pvals)
        lane = lax.iota(jnp.int32, 16)
        window = pvals[pl.ds((wid // 2) * 16, 16)]
        half = (wid % 2) * TPW

        def per_token(t, carry):
            p = jnp.sum(jnp.where(lane == half + t, window, 0))
            g_tok = wid * TPW + t
            for g in range(GPT):
                idx = p * F + (g * 16) + lane
                pltpu.async_copy(table_hbm.at[idx], buf, gsem).wait()
                pltpu.sync_copy(buf, out_hbm.at[pl.ds(g_tok * F + g * 16, 16)])
            return carry

        lax.fori_loop(0, TPW, per_token, 0)

    return gather_kernel


_gather = _make_kernel()


def kernel(prefix, embedding):
    p = prefix.reshape(-1).astype(jnp.int32)
    table = embedding.reshape(NUM_VIRTUAL_TOKENS * F, DC)
    out = _gather(p, table)
    return out.reshape(BATCH, NUM_VIRTUAL_TOKENS, ROW_DIM)


# speedup vs baseline: 1.1234x; 1.1234x over previous
import functools

import jax
import jax.numpy as jnp
from jax import lax
from jax.experimental import pallas as pl
from jax.experimental.pallas import tpu as pltpu
from jax.experimental.pallas import tpu_sc as plsc

BATCH = 4
NUM_VIRTUAL_TOKENS = 64
NUM_TOKENS = BATCH * NUM_VIRTUAL_TOKENS
ROW_DIM = 98304
F = 64
DC = ROW_DIM // F
NC, NS = 2, 16
NW = NC * NS
TPW = NUM_TOKENS // NW
GPT = F // 16


def _make_kernel():
    mesh = plsc.VectorSubcoreMesh(core_axis_name="c", subcore_axis_name="s")

    @functools.partial(
        pl.kernel,
        mesh=mesh,
        out_type=jax.ShapeDtypeStruct((NUM_TOKENS * F, DC), jnp.float32),
        compiler_params=pltpu.CompilerParams(needs_layout_passes=False),
        scratch_types=[
            pltpu.VMEM((NUM_TOKENS,), jnp.int32),
            pltpu.VMEM((16, DC), jnp.float32),
            pltpu.VMEM((16, DC), jnp.float32),
            pltpu.VMEM((16, DC), jnp.float32),
            pltpu.VMEM((16, DC), jnp.float32),
            pltpu.SemaphoreType.DMA,
            pltpu.SemaphoreType.DMA,
        ],
    )
    def gather_kernel(prefix_hbm, table_hbm, out_hbm, pvals,
                      buf0, buf1, buf2, buf3, gsem, wsem):
        wid = lax.axis_index("s") * NC + lax.axis_index("c")
        bufs = (buf0, buf1, buf2, buf3)
        pltpu.sync_copy(prefix_hbm, pvals)
        lane = lax.iota(jnp.int32, 16)
        window = pvals[pl.ds((wid // 2) * 16, 16)]
        half = (wid % 2) * TPW
        out_base = wid * TPW * F

        def pval(t):
            return jnp.sum(jnp.where(lane == half + t, window, 0))

        def wait_gather(b):
            pltpu.make_async_copy(table_hbm.at[lane], bufs[b], gsem).wait()

        def wait_write(b):
            pltpu.make_async_copy(bufs[b], out_hbm.at[pl.ds(0, 16)], wsem).wait()

        p0 = pval(0)
        for g in range(GPT):
            pltpu.async_copy(table_hbm.at[p0 * F + g * 16 + lane], bufs[g], gsem)
        wait_gather(0)
        pltpu.async_copy(bufs[0], out_hbm.at[pl.ds(out_base, 16)], wsem)

        def per_round(r, carry):
            p = pval(r)
            for b in range(GPT):
                wait_write(b)
                pltpu.async_copy(table_hbm.at[p * F + b * 16 + lane], bufs[b], gsem)
                bw = (b + 1) % 4
                t_w = r - 1 + (1 if b == 3 else 0)
                wait_gather(bw)
                pltpu.async_copy(
                    bufs[bw],
                    out_hbm.at[pl.ds(out_base + t_w * F + bw * 16, 16)],
                    wsem,
                )
            return carry

        lax.fori_loop(1, TPW, per_round, 0)

        for bw in (1, 2, 3):
            wait_gather(bw)
            pltpu.async_copy(
                bufs[bw],
                out_hbm.at[pl.ds(out_base + (TPW - 1) * F + bw * 16, 16)],
                wsem,
            )
        for b in range(4):
            wait_write(b)

    return gather_kernel


_gather = _make_kernel()


def kernel(prefix, embedding):
    p = prefix.reshape(-1).astype(jnp.int32)
    table = embedding.reshape(NUM_VIRTUAL_TOKENS * F, DC)
    out = _gather(p, table)
    return out.reshape(BATCH, NUM_VIRTUAL_TOKENS, ROW_DIM)


# speedup vs baseline: 1.1553x; 1.0284x over previous
import functools

import jax
import jax.numpy as jnp
from jax import lax
from jax.experimental import pallas as pl
from jax.experimental.pallas import tpu as pltpu
from jax.experimental.pallas import tpu_sc as plsc

BATCH = 4
NUM_VIRTUAL_TOKENS = 64
NUM_TOKENS = BATCH * NUM_VIRTUAL_TOKENS
ROW_DIM = 98304
F = 4
DC = ROW_DIM // F
NC, NS = 2, 16
NW = NC * NS
TPW = NUM_TOKENS // NW


def _make_kernel():
    mesh = plsc.VectorSubcoreMesh(core_axis_name="c", subcore_axis_name="s")

    @functools.partial(
        pl.kernel,
        mesh=mesh,
        out_type=jax.ShapeDtypeStruct((NUM_TOKENS * F, DC), jnp.float32),
        compiler_params=pltpu.CompilerParams(needs_layout_passes=False),
        scratch_types=[
            pltpu.VMEM((NUM_TOKENS,), jnp.int32),
            pltpu.VMEM((1, DC), jnp.float32),
            pltpu.VMEM((1, DC), jnp.float32),
            pltpu.VMEM((1, DC), jnp.float32),
            pltpu.VMEM((1, DC), jnp.float32),
            pltpu.SemaphoreType.DMA,
            pltpu.SemaphoreType.DMA,
        ],
    )
    def gather_kernel(prefix_hbm, table_hbm, out_hbm, pvals,
                      buf0, buf1, buf2, buf3, gsem, wsem):
        wid = lax.axis_index("s") * NC + lax.axis_index("c")
        bufs = (buf0, buf1, buf2, buf3)
        pltpu.sync_copy(prefix_hbm, pvals)
        lane = lax.iota(jnp.int32, 16)
        window = pvals[pl.ds((wid // 2) * 16, 16)]
        half = (wid % 2) * TPW
        out_base = wid * TPW * F

        def pval(t):
            return jnp.sum(jnp.where(lane == half + t, window, 0))

        def wait_gather(b):
            pltpu.make_async_copy(table_hbm.at[pl.ds(0, 1)], bufs[b], gsem).wait()

        def wait_write(b):
            pltpu.make_async_copy(bufs[b], out_hbm.at[pl.ds(0, 1)], wsem).wait()

        p0 = pval(0)
        for g in range(F):
            pltpu.async_copy(table_hbm.at[pl.ds(p0 * F + g, 1)], bufs[g], gsem)
        wait_gather(0)
        pltpu.async_copy(bufs[0], out_hbm.at[pl.ds(out_base, 1)], wsem)

        def per_round(r, carry):
            p = pval(r)
            for b in range(F):
                wait_write(b)
                pltpu.async_copy(table_hbm.at[pl.ds(p * F + b, 1)], bufs[b], gsem)
                bw = (b + 1) % 4
                t_w = r - 1 + (1 if b == 3 else 0)
                wait_gather(bw)
                pltpu.async_copy(
                    bufs[bw],
                    out_hbm.at[pl.ds(out_base + t_w * F + bw, 1)],
                    wsem,
                )
            return carry

        lax.fori_loop(1, TPW, per_round, 0)

        for bw in (1, 2, 3):
            wait_gather(bw)
            pltpu.async_copy(
                bufs[bw],
                out_hbm.at[pl.ds(out_base + (TPW - 1) * F + bw, 1)],
                wsem,
            )
        for b in range(4):
            wait_write(b)

    return gather_kernel


_gather = _make_kernel()


def kernel(prefix, embedding):
    p = prefix.reshape(-1).astype(jnp.int32)
    table = embedding.reshape(NUM_VIRTUAL_TOKENS * F, DC)
    out = _gather(p, table)
    return out.reshape(BATCH, NUM_VIRTUAL_TOKENS, ROW_DIM)


# speedup vs baseline: 2.2692x; 1.9642x over previous
import functools

import jax
import jax.numpy as jnp
from jax import lax
from jax.experimental import pallas as pl
from jax.experimental.pallas import tpu as pltpu
from jax.experimental.pallas import tpu_sc as plsc

BATCH = 4
NUM_VIRTUAL_TOKENS = 64
NUM_TOKENS = BATCH * NUM_VIRTUAL_TOKENS
ROW_DIM = 98304
W = 3072
NCH = ROW_DIM // W
NC, NS = 2, 16
NW = NC * NS
TPW = NUM_TOKENS // NW


def _make_kernel():
    mesh = plsc.VectorSubcoreMesh(core_axis_name="c", subcore_axis_name="s")

    @functools.partial(
        pl.kernel,
        mesh=mesh,
        out_type=jax.ShapeDtypeStruct((NUM_TOKENS, ROW_DIM), jnp.float32),
        compiler_params=pltpu.CompilerParams(needs_layout_passes=False),
        scratch_types=[
            pltpu.VMEM((NUM_TOKENS,), jnp.int32),
            pltpu.VMEM((TPW, W), jnp.float32),
            pltpu.VMEM((TPW, W), jnp.float32),
            pltpu.VMEM((TPW, W), jnp.float32),
            pltpu.VMEM((TPW, W), jnp.float32),
            pltpu.SemaphoreType.DMA,
            pltpu.SemaphoreType.DMA,
        ],
    )
    def gather_kernel(prefix_hbm, table_hbm, out_hbm, pvals,
                      buf0, buf1, buf2, buf3, gsem, wsem):
        wid = lax.axis_index("s") * NC + lax.axis_index("c")
        bufs = (buf0, buf1, buf2, buf3)
        pltpu.sync_copy(prefix_hbm, pvals)
        lane = lax.iota(jnp.int32, 16)
        window = pvals[pl.ds((wid // 2) * 16, 16)]
        half = (wid % 2) * TPW
        row0 = wid * TPW
        ps = [jnp.sum(jnp.where(lane == half + t, window, 0))
              for t in range(TPW)]

        def issue_gather(c, buf):
            for k in range(TPW):
                pltpu.async_copy(
                    table_hbm.at[pl.ds(ps[k], 1), pl.ds(c * W, W)],
                    buf.at[pl.ds(k, 1)],
                    gsem,
                )

        def issue_write(c, buf):
            pltpu.async_copy(
                buf, out_hbm.at[pl.ds(row0, TPW), pl.ds(c * W, W)], wsem)

        def wait_gather(b):
            pltpu.make_async_copy(
                table_hbm.at[pl.ds(0, TPW), pl.ds(0, W)], bufs[b], gsem).wait()

        def wait_write(b):
            pltpu.make_async_copy(
                bufs[b], out_hbm.at[pl.ds(0, TPW), pl.ds(0, W)], wsem).wait()

        for j in range(4):
            issue_gather(j, bufs[j])
        wait_gather(0)
        issue_write(0, bufs[0])

        def per_round(r, carry):
            for b in range(4):
                j = 4 * r + b
                wait_write(b)
                issue_gather(j, bufs[b])
                bw = (b + 1) % 4
                wait_gather(bw)
                issue_write(j - 3, bufs[bw])
            return carry

        lax.fori_loop(1, NCH // 4, per_round, 0)

        for jw in (NCH - 3, NCH - 2, NCH - 1):
            bw = jw % 4
            wait_gather(bw)
            issue_write(jw, bufs[bw])
        for b in range(4):
            wait_write(b)

    return gather_kernel


_gather = _make_kernel()


def kernel(prefix, embedding):
    p = prefix.reshape(-1).astype(jnp.int32)
    out = _gather(p, embedding)
    return out.reshape(BATCH, NUM_VIRTUAL_TOKENS, ROW_DIM)


# speedup vs baseline: 3.3939x; 1.4956x over previous
import functools

import jax
import jax.numpy as jnp
from jax import lax
from jax.experimental import pallas as pl
from jax.experimental.pallas import tpu as pltpu
from jax.experimental.pallas import tpu_sc as plsc

BATCH = 4
NUM_VIRTUAL_TOKENS = 64
NUM_TOKENS = BATCH * NUM_VIRTUAL_TOKENS
ROW_DIM = 98304
NC, NS = 2, 16
NW = NC * NS
CPW = ROW_DIM // NW
H = CPW // 2


def _make_kernel():
    mesh = plsc.VectorSubcoreMesh(core_axis_name="c", subcore_axis_name="s")

    @functools.partial(
        pl.kernel,
        mesh=mesh,
        out_type=jax.ShapeDtypeStruct((NUM_TOKENS, ROW_DIM), jnp.float32),
        compiler_params=pltpu.CompilerParams(needs_layout_passes=False),
        scratch_types=[
            pltpu.VMEM((NUM_TOKENS,), jnp.int32),
            pltpu.VMEM((NUM_VIRTUAL_TOKENS, H), jnp.float32),
            pltpu.SemaphoreType.DMA,
            pltpu.SemaphoreType.DMA,
        ],
    )
    def gather_kernel(prefix_hbm, table_hbm, out_hbm, pvals, tblk, gsem, wsem):
        wid = lax.axis_index("s") * NC + lax.axis_index("c")
        pltpu.sync_copy(prefix_hbm, pvals)
        lane = lax.iota(jnp.int32, 16)
        col0 = wid * CPW

        def drain_writes():
            for _ in range(4):
                pltpu.make_async_copy(
                    tblk, out_hbm.at[pl.ds(0, 64), pl.ds(0, H)], wsem).wait()

        for h in range(2):
            cbase = col0 + h * H
            pltpu.async_copy(
                table_hbm.at[:, pl.ds(cbase, H)], tblk, gsem).wait()

            def per_window(win, carry):
                wvec = pvals[pl.ds(win * 16, 16)]
                for l in range(16):
                    p = jnp.sum(jnp.where(lane == l, wvec, 0))
                    pltpu.async_copy(
                        tblk.at[pl.ds(p, 1)],
                        out_hbm.at[pl.ds(win * 16 + l, 1), pl.ds(cbase, H)],
                        wsem,
                    )
                return carry

            lax.fori_loop(0, NUM_TOKENS // 16, per_window, 0)
            drain_writes()

    return gather_kernel


_gather = _make_kernel()


def kernel(prefix, embedding):
    p = prefix.reshape(-1).astype(jnp.int32)
    out = _gather(p, embedding)
    return out.reshape(BATCH, NUM_VIRTUAL_TOKENS, ROW_DIM)
